# Initial kernel scaffold; baseline (speedup 1.0000x reference)
import jax
import jax.numpy as jnp
from jax.experimental import pallas as pl


def kernel(x, m):
    raise NotImplementedError("write your pallas kernel here")


# speedup vs baseline: 1.0518x; 1.0518x over previous
import functools

import jax
import jax.numpy as jnp
from jax import lax
from jax.experimental import pallas as pl
from jax.experimental.pallas import tpu as pltpu
from jax.experimental.pallas import tpu_sc as plsc

N, C, H, W = 4, 96, 384, 384
NPIX = N * H * W
NW = 32
PPW = NPIX // NW
P = 128
NCHUNK = PPW // P
GRP = P // 16
CV = C // 16

_mesh = plsc.VectorSubcoreMesh(core_axis_name="c", subcore_axis_name="s")


@functools.partial(
    pl.kernel,
    out_type=jax.ShapeDtypeStruct((NPIX, C), jnp.float32),
    mesh=_mesh,
    scratch_types=[
        pltpu.VMEM((P,), jnp.float32),
        pltpu.VMEM((P,), jnp.float32),
        pltpu.VMEM((P,), jnp.int32),
        pltpu.VMEM((P,), jnp.int32),
        pltpu.VMEM((P,), jnp.int32),
        pltpu.VMEM((P,), jnp.int32),
        pltpu.VMEM((P,), jnp.float32),
        pltpu.VMEM((P,), jnp.float32),
        pltpu.VMEM((P,), jnp.float32),
        pltpu.VMEM((P,), jnp.float32),
        pltpu.VMEM((P, C), jnp.float32),
        pltpu.VMEM((P, C), jnp.float32),
        pltpu.VMEM((P, C), jnp.float32),
        pltpu.VMEM((P, C), jnp.float32),
        pltpu.VMEM((P, C), jnp.float32),
        pltpu.SemaphoreType.DMA,
    ],
    compiler_params=pltpu.CompilerParams(use_tc_tiling_on_sc=False),
)
def _grid_sample_sc(xt, gx, gy, out, gxv, gyv, i00v, i01v, i10v, i11v,
                    w00v, w01v, w10v, w11v, r00, r01, r10, r11, outv, sem):
    cid = lax.axis_index("c")
    sid = lax.axis_index("s")
    wid = sid * 2 + cid
    base0 = wid * PPW
    img_base = (base0 // (H * W)) * (H * W)

    def chunk_body(ci, carry):
        base = base0 + ci * P
        pltpu.sync_copy(gx.at[pl.ds(base, P)], gxv)
        pltpu.sync_copy(gy.at[pl.ds(base, P)], gyv)

        for j in range(GRP):
            sl = pl.ds(j * 16, 16)
            ix = (gxv[sl] + 1.0) * 0.5 * (W - 1)
            iy = (gyv[sl] + 1.0) * 0.5 * (H - 1)
            ix0 = ix.astype(jnp.int32)
            iy0 = iy.astype(jnp.int32)
            wx1 = ix - ix0.astype(jnp.float32)
            wy1 = iy - iy0.astype(jnp.float32)
            wx0 = 1.0 - wx1
            wy0 = 1.0 - wy1
            ix1 = jnp.minimum(ix0 + 1, W - 1)
            iy1 = jnp.minimum(iy0 + 1, H - 1)
            row0 = iy0 * W + img_base
            row1 = iy1 * W + img_base
            i00v[sl] = row0 + ix0
            i01v[sl] = row0 + ix1
            i10v[sl] = row1 + ix0
            i11v[sl] = row1 + ix1
            w00v[sl] = wy0 * wx0
            w01v[sl] = wy0 * wx1
            w10v[sl] = wy1 * wx0
            w11v[sl] = wy1 * wx1

        d0 = pltpu.async_copy(xt.at[i00v], r00, sem)
        d1 = pltpu.async_copy(xt.at[i01v], r01, sem)
        d2 = pltpu.async_copy(xt.at[i10v], r10, sem)
        d3 = pltpu.async_copy(xt.at[i11v], r11, sem)
        d0.wait()
        d1.wait()
        d2.wait()
        d3.wait()

        def pix_group(j, c2):
            s = j * 16
            gsl = pl.ds(s, 16)
            wa = w00v[gsl]
            wb = w01v[gsl]
            wc = w10v[gsl]
            wd = w11v[gsl]
            for l in range(16):
                p = s + l
                a = wa[l]
                b = wb[l]
                c = wc[l]
                d = wd[l]
                for k in range(CV):
                    sl = pl.ds(k * 16, 16)
                    outv[p, sl] = (r00[p, sl] * a + r01[p, sl] * b
                                   + r10[p, sl] * c + r11[p, sl] * d)
            return c2

        lax.fori_loop(0, GRP, pix_group, 0)
        pltpu.sync_copy(outv, out.at[pl.ds(base, P)])
        return carry

    lax.fori_loop(0, NCHUNK, chunk_body, 0)


def kernel(x, m):
    xt = jnp.transpose(x, (0, 2, 3, 1)).reshape(NPIX, C)
    gx = m[..., 0].reshape(NPIX)
    gy = m[..., 1].reshape(NPIX)
    out_cl = _grid_sample_sc(xt, gx, gy)
    return out_cl.reshape(N, H, W, C).transpose(0, 3, 1, 2)
